# baseline (speedup 1.0000x reference)
import functools
import jax
import jax.numpy as jnp
from jax import lax
from jax.experimental import pallas as pl
from jax.experimental.pallas import tpu as pltpu
from jax.experimental.pallas import tpu_sc as plsc

N_CLASSES = 1000
BATCH = 16384
NW = 32
COLS = BATCH // NW
C_CHUNK = 40
N_CHUNK = N_CLASSES // C_CHUNK
VECS = COLS // 16

_mesh = plsc.VectorSubcoreMesh(core_axis_name="c", subcore_axis_name="s")


@functools.partial(
    pl.kernel,
    out_type=jax.ShapeDtypeStruct((N_CLASSES, BATCH), jnp.float32),
    mesh=_mesh,
    scratch_types=[
        pltpu.VMEM((COLS,), jnp.int32),
        pltpu.VMEM((C_CHUNK, COLS), jnp.float32),
        pltpu.VMEM((C_CHUNK, COLS), jnp.float32),
        pltpu.SemaphoreType.DMA,
        pltpu.SemaphoreType.DMA,
        pltpu.SemaphoreType.DMA,
    ],
    compiler_params=pltpu.CompilerParams(needs_layout_passes=False),
)
def _onehot_sc(idx_hbm, zeros_hbm, out_hbm, idx_v, buf0, buf1, sem0, sem1, semz):
    wid = lax.axis_index("s") * 2 + lax.axis_index("c")
    wbase = wid * COLS

    z0 = pltpu.async_copy(zeros_hbm, buf0, semz)
    z1 = pltpu.async_copy(zeros_hbm, buf1, semz)
    pltpu.sync_copy(idx_hbm.at[pl.ds(wbase, COLS)], idx_v)
    z0.wait()
    z1.wait()

    bufs = (buf0, buf1)
    sems = (sem0, sem1)
    one = jnp.full((16,), 1.0, jnp.float32)
    zero = jnp.full((16,), 0.0, jnp.float32)

    def scan_scatter(buf, set_lo, clear_lo):
        def body(v, _):
            idx = idx_v[pl.ds(v * 16, 16)]
            cols = lax.iota(jnp.int32, 16) + v * 16
            set_m = (idx >= set_lo) & (idx < set_lo + C_CHUNK)
            if clear_lo is None:
                rows = idx - set_lo
                mask = set_m
                vals = one
            else:
                clear_m = (idx >= clear_lo) & (idx < clear_lo + C_CHUNK)
                rows = idx - jnp.where(set_m, set_lo, clear_lo)
                mask = set_m | clear_m
                vals = jnp.where(set_m, one, zero)
            plsc.store_scatter(buf, [rows, cols], vals, mask=mask)
            return ()

        lax.fori_loop(0, VECS, body, (), unroll=2)

    copies = [None, None]
    for c in range(N_CHUNK):
        slot = c % 2
        if c >= 2:
            for cp in copies[slot]:
                cp.wait()
            scan_scatter(bufs[slot], c * C_CHUNK, (c - 2) * C_CHUNK)
        else:
            scan_scatter(bufs[slot], c * C_CHUNK, None)
        copies[slot] = [
            pltpu.async_copy(
                bufs[slot].at[pl.ds(8 * t, 8)],
                out_hbm.at[pl.ds(c * C_CHUNK + 8 * t, 8), pl.ds(wbase, COLS)],
                sems[slot],
            )
            for t in range(C_CHUNK // 8)
        ]
    for cp in copies[(N_CHUNK - 2) % 2] + copies[(N_CHUNK - 1) % 2]:
        cp.wait()


def kernel(inputs):
    idx = inputs.astype(jnp.int32)
    zeros = jnp.zeros((C_CHUNK, COLS), jnp.float32)
    out_t = _onehot_sc(idx, zeros)
    return out_t.T


# speedup vs baseline: 1.0713x; 1.0713x over previous
import functools
import jax
import jax.numpy as jnp
from jax import lax
from jax.experimental import pallas as pl
from jax.experimental.pallas import tpu as pltpu
from jax.experimental.pallas import tpu_sc as plsc

N_CLASSES = 1000
BATCH = 16384
NW = 32
COLS = BATCH // NW
C_CHUNK = 40
N_CHUNK = N_CLASSES // C_CHUNK
VECS = COLS // 16

_mesh = plsc.VectorSubcoreMesh(core_axis_name="c", subcore_axis_name="s")


@functools.partial(
    pl.kernel,
    out_type=jax.ShapeDtypeStruct((N_CLASSES, BATCH), jnp.float32),
    mesh=_mesh,
    scratch_types=[
        pltpu.VMEM((COLS,), jnp.int32),
        pltpu.VMEM((C_CHUNK, COLS), jnp.float32),
        pltpu.VMEM((C_CHUNK, COLS), jnp.float32),
        pltpu.SemaphoreType.DMA,
        pltpu.SemaphoreType.DMA,
        pltpu.SemaphoreType.DMA,
    ],
    compiler_params=pltpu.CompilerParams(needs_layout_passes=False),
)
def _onehot_sc(idx_hbm, zeros_hbm, out_hbm, idx_v, buf0, buf1, sem0, sem1, semz):
    wid = lax.axis_index("s") * 2 + lax.axis_index("c")
    wbase = wid * COLS

    z0 = pltpu.async_copy(zeros_hbm, buf0, semz)
    z1 = pltpu.async_copy(zeros_hbm, buf1, semz)
    pltpu.sync_copy(idx_hbm.at[pl.ds(wbase, COLS)], idx_v)
    z0.wait()
    z1.wait()

    bufs = (buf0, buf1)
    sems = (sem0, sem1)
    one = jnp.full((16,), 1.0, jnp.float32)
    zero = jnp.full((16,), 0.0, jnp.float32)

    def scan_scatter(buf, set_lo, clear_lo):
        def body(v, _):
            idx = idx_v[pl.ds(v * 16, 16)]
            cols = lax.iota(jnp.int32, 16) + v * 16
            set_m = (idx >= set_lo) & (idx < set_lo + C_CHUNK)
            if clear_lo is None:
                rows = idx - set_lo
                mask = set_m
                vals = one
            else:
                clear_m = (idx >= clear_lo) & (idx < clear_lo + C_CHUNK)
                rows = idx - jnp.where(set_m, set_lo, clear_lo)
                mask = set_m | clear_m
                vals = jnp.where(set_m, one, zero)
            plsc.store_scatter(buf, [rows, cols], vals, mask=mask)
            return ()

        lax.fori_loop(0, VECS, body, (), unroll=2)

    copies = [None, None]
    for c in range(N_CHUNK):
        slot = c % 2
        if c >= 2:
            copies[slot].wait()
            scan_scatter(bufs[slot], c * C_CHUNK, (c - 2) * C_CHUNK)
        else:
            scan_scatter(bufs[slot], c * C_CHUNK, None)
        copies[slot] = pltpu.async_copy(
            bufs[slot],
            out_hbm.at[pl.ds(c * C_CHUNK, C_CHUNK), pl.ds(wbase, COLS)],
            sems[slot],
        )
    copies[(N_CHUNK - 2) % 2].wait()
    copies[(N_CHUNK - 1) % 2].wait()


def kernel(inputs):
    idx = inputs.astype(jnp.int32)
    zeros = jnp.zeros((C_CHUNK, COLS), jnp.float32)
    out_t = _onehot_sc(idx, zeros)
    return out_t.T
